# baseline (speedup 1.0000x reference)
import jax
import jax.numpy as jnp
from jax import lax
from jax.experimental import pallas as pl
from jax.experimental.pallas import tpu as pltpu
from jax.experimental.pallas import tpu_sc as plsc

NC = 2
NS = 16
NW = NC * NS
LANES = 16
BATCH = 16384
D = 64
CHUNK = 128
B_PER_W = BATCH // NW
N_CHUNKS = B_PER_W // CHUNK
IDX_ROWS = BATCH // CHUNK
ROWS_PER_W = N_CHUNKS
COL_CHUNKS = D // LANES


def _sc_body(h_hbm, r_hbm, t_hbm, wen_hbm, wre_hbm, out_hbm,
             hidx, ridx, tidx, hbuf, rbuf, tbuf, acc,
             sem_i, sem_h, sem_r, sem_t):
    wid = lax.axis_index("s") * NC + lax.axis_index("c")
    acc[...] = jnp.zeros((LANES,), jnp.float32)
    base = wid * ROWS_PER_W
    ci_h = pltpu.async_copy(h_hbm.at[pl.ds(base, ROWS_PER_W)], hidx, sem_i)
    ci_r = pltpu.async_copy(r_hbm.at[pl.ds(base, ROWS_PER_W)], ridx, sem_h)
    ci_t = pltpu.async_copy(t_hbm.at[pl.ds(base, ROWS_PER_W)], tidx, sem_r)
    ci_h.wait()
    ci_r.wait()
    ci_t.wait()
    for j in range(N_CHUNKS):
        ch = pltpu.async_copy(wen_hbm.at[hidx.at[j]], hbuf, sem_h)
        cr = pltpu.async_copy(wre_hbm.at[ridx.at[j]], rbuf, sem_r)
        ct = pltpu.async_copy(wen_hbm.at[tidx.at[j]], tbuf, sem_t)
        ch.wait()
        cr.wait()
        ct.wait()

        @pl.loop(0, CHUNK)
        def _(i):
            for c in range(COL_CHUNKS):
                sl = pl.ds(c * LANES, LANES)
                v = hbuf[i, sl] + rbuf[i, sl] - tbuf[i, sl]
                acc[...] += v * v

    pltpu.sync_copy(acc, out_hbm.at[wid])


_sc_gather_reduce = pl.kernel(
    _sc_body,
    out_type=jax.ShapeDtypeStruct((NW, LANES), jnp.float32),
    mesh=plsc.VectorSubcoreMesh(core_axis_name="c", subcore_axis_name="s"),
    scratch_types=[
        pltpu.VMEM((ROWS_PER_W, CHUNK), jnp.int32),
        pltpu.VMEM((ROWS_PER_W, CHUNK), jnp.int32),
        pltpu.VMEM((ROWS_PER_W, CHUNK), jnp.int32),
        pltpu.VMEM((CHUNK, D), jnp.float32),
        pltpu.VMEM((CHUNK, D), jnp.float32),
        pltpu.VMEM((CHUNK, D), jnp.float32),
        pltpu.VMEM((LANES,), jnp.float32),
        pltpu.SemaphoreType.DMA,
        pltpu.SemaphoreType.DMA,
        pltpu.SemaphoreType.DMA,
        pltpu.SemaphoreType.DMA,
    ],
    compiler_params=pltpu.CompilerParams(use_tc_tiling_on_sc=False),
)


def _tc_reduce_body(p_ref, o_ref):
    o_ref[0, 0] = jnp.sum(p_ref[...])


def kernel(pos_h, pos_r, pos_t, neg_h, neg_r, neg_t, W_en, W_re):
    del neg_h, neg_r, neg_t
    h2 = pos_h.reshape(IDX_ROWS, CHUNK)
    r2 = pos_r.reshape(IDX_ROWS, CHUNK)
    t2 = pos_t.reshape(IDX_ROWS, CHUNK)
    partials = _sc_gather_reduce(h2, r2, t2, W_en, W_re)
    total = pl.pallas_call(
        _tc_reduce_body,
        out_shape=jax.ShapeDtypeStruct((1, 1), jnp.float32),
        out_specs=pl.BlockSpec(memory_space=pltpu.SMEM),
    )(partials)
    return total[0, 0]
